# Initial kernel scaffold; baseline (speedup 1.0000x reference)
import jax
import jax.numpy as jnp
from jax.experimental import pallas as pl


def kernel(x, edge_index, W_rel1, W_root1, b1, W_rel2, W_root2, b2, W_rel3, W_root3, b3):
    raise NotImplementedError("write your pallas kernel here")


# speedup vs baseline: 3.9418x; 3.9418x over previous
import functools

import jax
import jax.numpy as jnp
from jax import lax
from jax.experimental import pallas as pl
from jax.experimental.pallas import tpu as pltpu
from jax.experimental.pallas import tpu_sc as plsc

NC = 2
NS = 16
NW = NC * NS
CHUNK = 128


def _sc_aggregate(h, src, dst, zeros, n, acc_rows, chunks_per_tile):
    d = h.shape[1]
    edges_per_tile = chunks_per_tile * CHUNK
    zstripe = acc_rows // NS
    ostripe = (n // NS) & ~7
    tail = n - NS * ostripe
    mesh = plsc.VectorSubcoreMesh(core_axis_name="c", subcore_axis_name="s")

    @functools.partial(
        pl.kernel,
        out_type=jax.ShapeDtypeStruct((2 * n, d), jnp.float32),
        mesh=mesh,
        scratch_types=[
            pltpu.VMEM((CHUNK,), jnp.int32),
            pltpu.VMEM((CHUNK,), jnp.int32),
            pltpu.VMEM((CHUNK, d), jnp.float32),
            pltpu.VMEM_SHARED((acc_rows, d), jnp.float32),
            pltpu.SemaphoreType.DMA,
        ],
    )
    def k(h_hbm, src_hbm, dst_hbm, z_hbm, out_hbm, src_v, dst_v, rows_v, acc, sem):
        c = lax.axis_index("c")
        s = lax.axis_index("s")
        wid = s * NC + c

        pltpu.sync_copy(z_hbm.at[pl.ds(s * zstripe, zstripe)],
                        acc.at[pl.ds(s * zstripe, zstripe)])
        plsc.subcore_barrier()

        base = wid * edges_per_tile

        @pl.loop(0, chunks_per_tile)
        def _(i):
            off = base + i * CHUNK
            pltpu.sync_copy(src_hbm.at[pl.ds(off, CHUNK)], src_v)
            pltpu.sync_copy(dst_hbm.at[pl.ds(off, CHUNK)], dst_v)
            pltpu.async_copy(h_hbm.at[src_v], rows_v, sem).wait()
            pltpu.sync_copy(rows_v, acc.at[dst_v], add=True)

        plsc.subcore_barrier()
        pltpu.sync_copy(acc.at[pl.ds(s * ostripe, ostripe)],
                        out_hbm.at[pl.ds(c * n + s * ostripe, ostripe)])
        if tail:
            @pl.when(s == NS - 1)
            def _():
                pltpu.sync_copy(acc.at[pl.ds(NS * ostripe, tail)],
                                out_hbm.at[pl.ds(c * n + NS * ostripe, tail)])

    return k(h, src, dst, zeros)


def _tc_combine(p0, p1, h, wr_t, wo_t, b2d, relu, blk):
    n, d = h.shape

    def body(p0_ref, p1_ref, h_ref, wr_ref, wo_ref, b_ref, o_ref):
        agg = p0_ref[...] + p1_ref[...]
        out = jnp.dot(agg, wr_ref[...], preferred_element_type=jnp.float32)
        out = out + jnp.dot(h_ref[...], wo_ref[...],
                            preferred_element_type=jnp.float32)
        out = out + b_ref[...]
        if relu:
            out = jnp.maximum(out, 0.0)
        o_ref[...] = out

    return pl.pallas_call(
        body,
        grid=(n // blk,),
        in_specs=[
            pl.BlockSpec((blk, d), lambda i: (i, 0)),
            pl.BlockSpec((blk, d), lambda i: (i, 0)),
            pl.BlockSpec((blk, d), lambda i: (i, 0)),
            pl.BlockSpec((d, d), lambda i: (0, 0)),
            pl.BlockSpec((d, d), lambda i: (0, 0)),
            pl.BlockSpec((1, d), lambda i: (0, 0)),
        ],
        out_specs=pl.BlockSpec((blk, d), lambda i: (i, 0)),
        out_shape=jax.ShapeDtypeStruct((n, d), jnp.float32),
    )(p0, p1, h, wr_t, wo_t, b2d)


def kernel(x, edge_index, W_rel1, W_root1, b1, W_rel2, W_root2, b2,
           W_rel3, W_root3, b3):
    n, d = x.shape
    e = edge_index.shape[1]

    tile_span = NW * CHUNK
    chunks_per_tile = -(-e // tile_span)
    e_pad = chunks_per_tile * tile_span
    acc_rows = -(-(n + 1) // (NS * 8)) * (NS * 8)

    src = edge_index[0].astype(jnp.int32)
    dst = edge_index[1].astype(jnp.int32)
    pad = e_pad - e
    if pad:
        src = jnp.concatenate([src, jnp.zeros((pad,), jnp.int32)])
        dst = jnp.concatenate([dst, jnp.full((pad,), n, jnp.int32)])
    zeros = jnp.zeros((acc_rows, d), jnp.float32)

    blk = 1000 if n % 1000 == 0 else 8
    layers = [
        (W_rel1, W_root1, b1, True),
        (W_rel2, W_root2, b2, True),
        (W_rel3, W_root3, b3, False),
    ]
    h = x
    for wr, wo, b, relu in layers:
        parts = _sc_aggregate(h, src, dst, zeros, n, acc_rows, chunks_per_tile)
        h = _tc_combine(parts[:n], parts[n:], h, wr.T, wo.T,
                        b.reshape(1, d), relu, blk)
    return h
